# baseline (speedup 1.0000x reference)
import functools

import jax
import jax.numpy as jnp
from jax.experimental import pallas as pl


def _body(E, R, x_ref, wg_ref, a_ref, b_ref, o_ref):
    x = x_ref[...]
    g = jnp.dot(x, wg_ref[...], preferred_element_type=jnp.float32)
    lane_e = jax.lax.broadcasted_iota(jnp.int32, g.shape, 1)
    m1 = jnp.max(g, axis=1, keepdims=True)
    idx1 = jnp.min(jnp.where(g == m1, lane_e, E), axis=1, keepdims=True)
    g2 = jnp.where(lane_e == idx1, -jnp.inf, g)
    m2 = jnp.max(g2, axis=1, keepdims=True)
    idx2 = jnp.min(jnp.where(g2 == m2, lane_e, E), axis=1, keepdims=True)
    t = jnp.exp(m2 - m1)
    w1 = 1.0 / (1.0 + t)
    w2 = t / (1.0 + t)

    p = jnp.dot(x, a_ref[...], preferred_element_type=jnp.float32)
    e_of_lane = jax.lax.broadcasted_iota(jnp.int32, p.shape, 1) // R
    wfull = jnp.where(e_of_lane == idx1, w1, 0.0) + jnp.where(
        e_of_lane == idx2, w2, 0.0)
    o_ref[...] = jnp.dot(p * wfull, b_ref[...],
                         preferred_element_type=jnp.float32)


def kernel(inputs, Wg, A, Bm):
    Bsz, S, D = inputs.shape
    E, _, R = A.shape
    T = Bsz * S
    x = inputs.reshape(T, D)
    a_all = jnp.transpose(A, (1, 0, 2)).reshape(D, E * R)
    b_all = Bm.reshape(E * R, D)

    TB = 256
    out = pl.pallas_call(
        functools.partial(_body, E, R),
        grid=(T // TB,),
        in_specs=[
            pl.BlockSpec((TB, D), lambda i: (i, 0)),
            pl.BlockSpec((D, E), lambda i: (0, 0)),
            pl.BlockSpec((D, E * R), lambda i: (0, 0)),
            pl.BlockSpec((E * R, D), lambda i: (0, 0)),
        ],
        out_specs=pl.BlockSpec((TB, D), lambda i: (i, 0)),
        out_shape=jax.ShapeDtypeStruct((T, D), jnp.float32),
    )(x, Wg, a_all, b_all)
    return out.reshape(Bsz, S, D)


# speedup vs baseline: 1.1141x; 1.1141x over previous
import functools

import jax
import jax.numpy as jnp
from jax.experimental import pallas as pl


def _body(E, R, x_ref, wg_ref, a_ref, b_ref, o_ref):
    x = x_ref[...]
    g = jnp.dot(x, wg_ref[...], preferred_element_type=jnp.float32)
    lane_e = jax.lax.broadcasted_iota(jnp.int32, g.shape, 1)
    m1 = jnp.max(g, axis=1, keepdims=True)
    idx1 = jnp.min(jnp.where(g == m1, lane_e, E), axis=1, keepdims=True)
    g2 = jnp.where(lane_e == idx1, -jnp.inf, g)
    m2 = jnp.max(g2, axis=1, keepdims=True)
    idx2 = jnp.min(jnp.where(g2 == m2, lane_e, E), axis=1, keepdims=True)
    t = jnp.exp(m2 - m1)
    w1 = 1.0 / (1.0 + t)
    w2 = t / (1.0 + t)

    p = jnp.dot(x, a_ref[...], preferred_element_type=jnp.float32)
    e_of_lane = jax.lax.broadcasted_iota(jnp.int32, p.shape, 1) // R
    wfull = jnp.where(e_of_lane == idx1, w1, 0.0) + jnp.where(
        e_of_lane == idx2, w2, 0.0)
    o_ref[...] = jnp.dot(p * wfull, b_ref[...],
                         preferred_element_type=jnp.float32)


def kernel(inputs, Wg, A, Bm):
    Bsz, S, D = inputs.shape
    E, _, R = A.shape
    T = Bsz * S
    x = inputs.reshape(T, D)
    a_all = jnp.transpose(A, (1, 0, 2)).reshape(D, E * R)
    b_all = Bm.reshape(E * R, D)

    TB = 1024
    out = pl.pallas_call(
        functools.partial(_body, E, R),
        grid=(T // TB,),
        in_specs=[
            pl.BlockSpec((TB, D), lambda i: (i, 0)),
            pl.BlockSpec((D, E), lambda i: (0, 0)),
            pl.BlockSpec((D, E * R), lambda i: (0, 0)),
            pl.BlockSpec((E * R, D), lambda i: (0, 0)),
        ],
        out_specs=pl.BlockSpec((TB, D), lambda i: (i, 0)),
        out_shape=jax.ShapeDtypeStruct((T, D), jnp.float32),
    )(x, Wg, a_all, b_all)
    return out.reshape(Bsz, S, D)
